# baseline (speedup 1.0000x reference)
import jax
import jax.numpy as jnp
from jax.experimental import pallas as pl

EMB = 16
H = 4
D_ATT = 16
B = 4096
M = 66
ROWS = B * M
G = H * B
NB = 32
PR = 66 * 128


def _proj_body(x_ref, wq_ref, wk_ref, wv_ref, wr_ref, q_ref, k_ref, v_ref, r_ref):
    x = x_ref[...]
    q_ref[...] = jnp.dot(x, wq_ref[...], preferred_element_type=jnp.float32)
    k_ref[...] = jnp.dot(x, wk_ref[...], preferred_element_type=jnp.float32)
    v_ref[...] = jnp.dot(x, wv_ref[...], preferred_element_type=jnp.float32)
    r_ref[...] = jnp.dot(x, wr_ref[...], preferred_element_type=jnp.float32)


def _attn_body(q0, q1, q2, q3, k0, k1, k2, k3, v0, v1, v2, v3, xr_ref, o_ref):
    qs = (q0, q1, q2, q3)
    ks = (k0, k1, k2, k3)
    vs = (v0, v1, v2, v3)

    def body(t, _):
        acc = jnp.zeros((M, D_ATT), dtype=jnp.float32)
        for h in range(H):
            q = qs[h][t]
            k = ks[h][t]
            v = vs[h][t]
            s = jnp.dot(q, k, preferred_element_type=jnp.float32)
            smax = jnp.max(s, axis=1, keepdims=True)
            e = jnp.exp(s - smax)
            den = jnp.sum(e, axis=1, keepdims=True)
            att = e / den
            acc = acc + jnp.dot(att, v, preferred_element_type=jnp.float32)
        o_ref[t] = jax.nn.relu(acc * (1.0 / H) + xr_ref[t])
        return 0

    jax.lax.fori_loop(0, NB, body, 0)


def _attention_layer(xf, p):
    nsteps = ROWS // PR
    q2, k2, v2, xr = pl.pallas_call(
        _proj_body,
        grid=(nsteps,),
        in_specs=[
            pl.BlockSpec((PR, EMB), lambda i: (i, 0)),
            pl.BlockSpec((EMB, H * D_ATT), lambda i: (0, 0)),
            pl.BlockSpec((EMB, H * D_ATT), lambda i: (0, 0)),
            pl.BlockSpec((EMB, H * D_ATT), lambda i: (0, 0)),
            pl.BlockSpec((EMB, D_ATT), lambda i: (0, 0)),
        ],
        out_specs=[
            pl.BlockSpec((PR, H * D_ATT), lambda i: (i, 0)),
            pl.BlockSpec((PR, H * D_ATT), lambda i: (i, 0)),
            pl.BlockSpec((PR, H * D_ATT), lambda i: (i, 0)),
            pl.BlockSpec((PR, EMB), lambda i: (i, 0)),
        ],
        out_shape=[
            jax.ShapeDtypeStruct((ROWS, H * D_ATT), jnp.float32),
            jax.ShapeDtypeStruct((ROWS, H * D_ATT), jnp.float32),
            jax.ShapeDtypeStruct((ROWS, H * D_ATT), jnp.float32),
            jax.ShapeDtypeStruct((ROWS, EMB), jnp.float32),
        ],
    )(xf, p['wq'].T, p['wk'].T, p['wv'].T, p['wr'].T)

    q3 = q2.reshape(G, M, D_ATT)
    k3 = k2.reshape(G, D_ATT, M)
    v3 = v2.reshape(G, M, D_ATT)
    xr3 = xr.reshape(B, M, EMB)

    nblk = B // NB
    qspec = [pl.BlockSpec((NB, M, D_ATT), (lambda i, h=h: (h * nblk + i, 0, 0)))
             for h in range(H)]
    kspec = [pl.BlockSpec((NB, D_ATT, M), (lambda i, h=h: (h * nblk + i, 0, 0)))
             for h in range(H)]
    vspec = [pl.BlockSpec((NB, M, D_ATT), (lambda i, h=h: (h * nblk + i, 0, 0)))
             for h in range(H)]
    out3 = pl.pallas_call(
        _attn_body,
        grid=(nblk,),
        in_specs=qspec + kspec + vspec + [
            pl.BlockSpec((NB, M, EMB), lambda i: (i, 0, 0)),
        ],
        out_specs=pl.BlockSpec((NB, M, EMB), lambda i: (i, 0, 0)),
        out_shape=jax.ShapeDtypeStruct((B, M, EMB), jnp.float32),
    )(q3, q3, q3, q3, k3, k3, k3, k3, v3, v3, v3, v3, xr3)
    return out3.reshape(ROWS, EMB)


def _va_body(xd_ref, vw_ref, vb_ref, aw_ref, ab_ref, o_ref):
    xd = xd_ref[...]
    va = jnp.dot(xd[:, :128], vw_ref[...], preferred_element_type=jnp.float32)
    aa = jnp.dot(xd[:, 128:], aw_ref[...], preferred_element_type=jnp.float32)
    o_ref[...] = jnp.concatenate([va + vb_ref[...], aa + ab_ref[...]], axis=1)


def _final_body(x_ref, w_ref, b_ref, o_ref):
    o_ref[...] = jnp.sum(x_ref[...] * w_ref[...], axis=1, keepdims=True) + b_ref[0, 0]


def kernel(x, emb1, pair_tables, emb3, title_table, video_W, video_b,
           audio_W, audio_b, att1, att2, lin_W, lin_b):
    xi = x.astype(jnp.int32)
    feats = []
    for i in range(9):
        feats.append(emb1[i][xi[:, i]][:, None, :])
    inc = 0
    for i in range(9):
        for j in range(i, 9):
            t1, t2 = pair_tables[inc]
            feats.append((t1[xi[:, i]] * t2[xi[:, j]])[:, None, :])
            inc += 1
    for i in range(9):
        feats.append(emb3[i][xi[:, i + 9]][:, None, :])
    feats.append(jnp.mean(title_table[xi[:, 18:28]], axis=1)[:, None, :])
    fields = jnp.concatenate(feats, axis=1)

    va = pl.pallas_call(
        _va_body,
        out_shape=jax.ShapeDtypeStruct((B, 2 * EMB), jnp.float32),
    )(x[:, 28:284], video_W.T, video_b[None, :], audio_W.T, audio_b[None, :])

    xf = jnp.concatenate([fields, va.reshape(B, 2, EMB)], axis=1).reshape(ROWS, EMB)

    xf = _attention_layer(xf, att1)
    xf = _attention_layer(xf, att2)
    xf = _attention_layer(xf, att2)

    return pl.pallas_call(
        _final_body,
        out_shape=jax.ShapeDtypeStruct((B, 1), jnp.float32),
    )(xf.reshape(B, M * EMB), lin_W, lin_b[None, :])
